# Initial kernel scaffold; baseline (speedup 1.0000x reference)
import functools

import jax
import jax.numpy as jnp
from jax import lax
from jax.experimental import pallas as pl
from jax.experimental.pallas import tpu as pltpu
from jax.experimental.pallas import tpu_sc as plsc

N, DIM, PD, K = 4096, 256, 3, 16
PDP = 16
GCOLS = 2 * DIM + PDP
KNN_BLK = 256
MAIN_BLK = 128
NW = 32
ROWS_PER_W = (N * K) // NW
GCHUNK = 128
INV_SQRT_D = 1.0 / 16.0


def _prep_body(x_ref, pospad_ref, Wq_ref, Wk_ref, Wv_ref, Wm1_ref, Wp2_ref,
               bm1_ref, bp2_ref, A_ref, G_ref, Wmix_ref, c0_ref):
    x = x_ref[...]
    Wm1 = Wm1_ref[...]
    A_ref[...] = x @ (Wq_ref[...] @ Wm1)
    kx = x @ (Wk_ref[...] @ Wm1)
    v = x @ Wv_ref[...]
    G_ref[...] = jnp.concatenate([kx, v, pospad_ref[...]], axis=1)
    Wp2 = Wp2_ref[...]
    Wmix_ref[...] = jnp.concatenate([Wp2 @ Wm1, Wp2], axis=1)
    c0_ref[...] = bp2_ref[...] @ Wm1 + bm1_ref[...]


def _prep(x, pospad, Wq, Wk, Wv, Wm1, Wp2, bm1, bp2):
    return pl.pallas_call(
        _prep_body,
        out_shape=(
            jax.ShapeDtypeStruct((N, DIM), jnp.float32),
            jax.ShapeDtypeStruct((N, GCOLS), jnp.float32),
            jax.ShapeDtypeStruct((DIM, 2 * DIM), jnp.float32),
            jax.ShapeDtypeStruct((1, DIM), jnp.float32),
        ),
    )(x, pospad, Wq, Wk, Wv, Wm1, Wp2, bm1, bp2)


def _knn_body(posb_ref, pospadT_ref, ind_ref):
    posb = posb_ref[...]
    pT = pospadT_ref[...]
    sqb = jnp.sum(posb * posb, axis=1, keepdims=True)
    sqf = jnp.sum(pT * pT, axis=0, keepdims=True)
    d2 = sqb + sqf - 2.0 * jnp.dot(posb, pT, preferred_element_type=jnp.float32)
    iota = lax.broadcasted_iota(jnp.int32, (KNN_BLK, N), 1)
    cols = []
    for _ in range(K):
        m = jnp.min(d2, axis=1, keepdims=True)
        am = jnp.min(jnp.where(d2 == m, iota, N), axis=1, keepdims=True)
        cols.append(am)
        d2 = jnp.where(iota == am, jnp.inf, d2)
    ind_ref[...] = jnp.concatenate(cols, axis=1)


def _knn(pospad, pospadT):
    return pl.pallas_call(
        _knn_body,
        grid=(N // KNN_BLK,),
        in_specs=[
            pl.BlockSpec((KNN_BLK, PDP), lambda i: (i, 0)),
            pl.BlockSpec((PDP, N), lambda i: (0, 0)),
        ],
        out_specs=pl.BlockSpec((KNN_BLK, K), lambda i: (i, 0)),
        out_shape=jax.ShapeDtypeStruct((N, K), jnp.int32),
    )(pospad, pospadT)


def _gather_body(G_hbm, ind_hbm, Gg_hbm, idx_v, rows_v, sem):
    wid = lax.axis_index("s") * 2 + lax.axis_index("c")
    base = wid * ROWS_PER_W
    pltpu.sync_copy(ind_hbm.at[pl.ds(base, ROWS_PER_W)], idx_v)
    for c in range(ROWS_PER_W // GCHUNK):
        pltpu.async_copy(
            G_hbm.at[idx_v.at[pl.ds(c * GCHUNK, GCHUNK)]], rows_v, sem
        ).wait()
        pltpu.sync_copy(rows_v, Gg_hbm.at[pl.ds(base + c * GCHUNK, GCHUNK)])


def _gather(G, ind_flat):
    mesh = plsc.VectorSubcoreMesh(core_axis_name="c", subcore_axis_name="s")
    run = pl.kernel(
        _gather_body,
        mesh=mesh,
        out_type=jax.ShapeDtypeStruct((N * K, GCOLS), jnp.float32),
        scratch_types=[
            pltpu.VMEM((ROWS_PER_W,), jnp.int32),
            pltpu.VMEM((GCHUNK, GCOLS), jnp.float32),
            pltpu.SemaphoreType.DMA,
        ],
    )
    return run(G, ind_flat)


def _main_body(A_ref, x_ref, posb_ref, Gg_ref, Wp1_ref, bp1_ref, Wmix_ref,
               c0_ref, Wm2_ref, bm2_ref, bp2_ref, Wf_ref, bf_ref, out_ref):
    Gg = Gg_ref[...]
    Bg = Gg[:, :DIM]
    Vg = Gg[:, DIM:2 * DIM]
    posg = Gg[:, 2 * DIM:]
    posb = posb_ref[...]
    relb = jnp.broadcast_to(
        posb.reshape(MAIN_BLK, 1, PDP), (MAIN_BLK, K, PDP)
    ).reshape(MAIN_BLK * K, PDP)
    rel = relb - posg
    h = jnp.maximum(
        jnp.dot(rel, Wp1_ref[...], preferred_element_type=jnp.float32)
        + bp1_ref[...], 0.0)
    pemix = jnp.dot(h, Wmix_ref[...], preferred_element_type=jnp.float32)
    a = jnp.broadcast_to(
        A_ref[...].reshape(MAIN_BLK, 1, DIM), (MAIN_BLK, K, DIM)
    ).reshape(MAIN_BLK * K, DIM)
    u = a - Bg + pemix[:, :DIM] + c0_ref[...]
    t = jnp.dot(jnp.maximum(u, 0.0), Wm2_ref[...],
                preferred_element_type=jnp.float32) + bm2_ref[...]
    l3 = (t * INV_SQRT_D).reshape(MAIN_BLK, K, DIM)
    mx = jnp.max(l3, axis=1, keepdims=True)
    e = jnp.exp(l3 - mx)
    s = e / jnp.sum(e, axis=1, keepdims=True)
    z = (Vg + pemix[:, DIM:] + bp2_ref[...]).reshape(MAIN_BLK, K, DIM)
    o = jnp.sum(s * z, axis=1)
    out_ref[...] = (
        jnp.dot(o, Wf_ref[...], preferred_element_type=jnp.float32)
        + bf_ref[...] + x_ref[...])


def _main(A, x, pospad, Gg, Wp1pad, bp1, Wmix, c0, Wm2, bm2, bp2, Wf, bf):
    nb = N // MAIN_BLK
    full = lambda r, c: pl.BlockSpec((r, c), lambda i: (0, 0))
    return pl.pallas_call(
        _main_body,
        grid=(nb,),
        in_specs=[
            pl.BlockSpec((MAIN_BLK, DIM), lambda i: (i, 0)),
            pl.BlockSpec((MAIN_BLK, DIM), lambda i: (i, 0)),
            pl.BlockSpec((MAIN_BLK, PDP), lambda i: (i, 0)),
            pl.BlockSpec((MAIN_BLK * K, GCOLS), lambda i: (i, 0)),
            full(PDP, DIM),
            full(1, DIM),
            full(DIM, 2 * DIM),
            full(1, DIM),
            full(DIM, DIM),
            full(1, DIM),
            full(1, DIM),
            full(DIM, DIM),
            full(1, DIM),
        ],
        out_specs=pl.BlockSpec((MAIN_BLK, DIM), lambda i: (i, 0)),
        out_shape=jax.ShapeDtypeStruct((N, DIM), jnp.float32),
    )(A, x, pospad, Gg, Wp1pad, bp1, Wmix, c0, Wm2, bm2, bp2, Wf, bf)


def kernel(x, pos, Wq, Wk, Wv, Wm1, bm1, Wm2, bm2, Wp1, bp1, Wp2, bp2, Wf, bf):
    pospad = jnp.pad(pos, ((0, 0), (0, PDP - PD)))
    pospadT = pospad.T
    Wp1pad = jnp.pad(Wp1, ((0, PDP - PD), (0, 0)))
    r = lambda b: b.reshape(1, DIM)

    A, G, Wmix, c0 = _prep(x, pospad, Wq, Wk, Wv, Wm1, Wp2, r(bm1), r(bp2))
    ind = _knn(pospad, pospadT)
    Gg = _gather(G, ind.reshape(N * K))
    return _main(A, x, pospad, Gg, Wp1pad, r(bp1), Wmix, c0, Wm2, r(bm2),
                 r(bp2), Wf, r(bf))


# speedup vs baseline: 7.5396x; 7.5396x over previous
import functools

import jax
import jax.numpy as jnp
from jax import lax
from jax.experimental import pallas as pl
from jax.experimental.pallas import tpu as pltpu
from jax.experimental.pallas import tpu_sc as plsc

N, DIM, PD, K = 4096, 256, 3, 16
PDP = 16
PPAD = 128
GCOLS = 2 * DIM + PPAD
KNN_BLK = 256
MAIN_BLK = 128
NW = 32
ROWS_PER_W = (N * K) // NW
GCHUNK = 128
INV_SQRT_D = 1.0 / 16.0


def _prep_body(x_ref, pospad_ref, Wq_ref, Wk_ref, Wv_ref, Wm1_ref, Wp2_ref,
               bm1_ref, bp2_ref, A_ref, G_ref, Wmix_ref, c0_ref):
    x = x_ref[...]
    Wm1 = Wm1_ref[...]
    A_ref[...] = x @ (Wq_ref[...] @ Wm1)
    kx = x @ (Wk_ref[...] @ Wm1)
    v = x @ Wv_ref[...]
    G_ref[...] = jnp.concatenate([kx, v, pospad_ref[...]], axis=1)
    Wp2 = Wp2_ref[...]
    Wmix_ref[...] = jnp.concatenate([Wp2 @ Wm1, Wp2], axis=1)
    c0_ref[...] = bp2_ref[...] @ Wm1 + bm1_ref[...]


def _prep(x, pospad, Wq, Wk, Wv, Wm1, Wp2, bm1, bp2):
    return pl.pallas_call(
        _prep_body,
        out_shape=(
            jax.ShapeDtypeStruct((N, DIM), jnp.float32),
            jax.ShapeDtypeStruct((N, GCOLS), jnp.float32),
            jax.ShapeDtypeStruct((DIM, 2 * DIM), jnp.float32),
            jax.ShapeDtypeStruct((1, DIM), jnp.float32),
        ),
    )(x, pospad, Wq, Wk, Wv, Wm1, Wp2, bm1, bp2)


def _knn_body(posb_ref, pospadT_ref, ind_ref):
    posb = posb_ref[...]
    pT = pospadT_ref[...]
    sqb = jnp.sum(posb * posb, axis=1, keepdims=True)
    sqf = jnp.sum(pT * pT, axis=0, keepdims=True)
    d2 = sqb + sqf - 2.0 * jnp.dot(posb, pT, preferred_element_type=jnp.float32)
    iota = lax.broadcasted_iota(jnp.int32, (KNN_BLK, N), 1)
    cols = []
    for _ in range(K):
        m = jnp.min(d2, axis=1, keepdims=True)
        am = jnp.min(jnp.where(d2 == m, iota, N), axis=1, keepdims=True)
        cols.append(am)
        d2 = jnp.where(iota == am, jnp.inf, d2)
    ind_ref[...] = jnp.concatenate(cols, axis=1)


def _knn(pospad, pospadT):
    return pl.pallas_call(
        _knn_body,
        grid=(N // KNN_BLK,),
        in_specs=[
            pl.BlockSpec((KNN_BLK, PDP), lambda i: (i, 0)),
            pl.BlockSpec((PDP, N), lambda i: (0, 0)),
        ],
        out_specs=pl.BlockSpec((KNN_BLK, K), lambda i: (i, 0)),
        out_shape=jax.ShapeDtypeStruct((N, K), jnp.int32),
    )(pospad, pospadT)


def _gather_body(G_hbm, ind_hbm, Gg_hbm, idx_v, rows_v, sem):
    wid = lax.axis_index("s") * 2 + lax.axis_index("c")
    base = wid * ROWS_PER_W
    pltpu.sync_copy(ind_hbm.at[pl.ds(base, ROWS_PER_W)], idx_v)
    for c in range(ROWS_PER_W // GCHUNK):
        pltpu.async_copy(
            G_hbm.at[idx_v.at[pl.ds(c * GCHUNK, GCHUNK)]], rows_v, sem
        ).wait()
        pltpu.sync_copy(rows_v, Gg_hbm.at[pl.ds(base + c * GCHUNK, GCHUNK)])


def _gather(G, ind_flat):
    mesh = plsc.VectorSubcoreMesh(core_axis_name="c", subcore_axis_name="s")
    run = pl.kernel(
        _gather_body,
        mesh=mesh,
        out_type=jax.ShapeDtypeStruct((N * K, GCOLS), jnp.float32),
        scratch_types=[
            pltpu.VMEM((ROWS_PER_W,), jnp.int32),
            pltpu.VMEM((GCHUNK, GCOLS), jnp.float32),
            pltpu.SemaphoreType.DMA,
        ],
    )
    return run(G, ind_flat)


def _main_body(A_ref, x_ref, posb_ref, Gg_ref, Wp1_ref, bp1_ref, Wmix_ref,
               c0_ref, Wm2_ref, bm2_ref, bp2_ref, Wf_ref, bf_ref, out_ref):
    Gg = Gg_ref[...]
    Bg = Gg[:, :DIM]
    Vg = Gg[:, DIM:2 * DIM]
    posg = Gg[:, 2 * DIM:]
    posb = posb_ref[...]
    relb = jnp.broadcast_to(
        posb.reshape(MAIN_BLK, 1, PPAD), (MAIN_BLK, K, PPAD)
    ).reshape(MAIN_BLK * K, PPAD)
    rel = relb - posg
    h = jnp.maximum(
        jnp.dot(rel, Wp1_ref[...], preferred_element_type=jnp.float32)
        + bp1_ref[...], 0.0)
    pemix = jnp.dot(h, Wmix_ref[...], preferred_element_type=jnp.float32)
    a = jnp.broadcast_to(
        A_ref[...].reshape(MAIN_BLK, 1, DIM), (MAIN_BLK, K, DIM)
    ).reshape(MAIN_BLK * K, DIM)
    u = a - Bg + pemix[:, :DIM] + c0_ref[...]
    t = jnp.dot(jnp.maximum(u, 0.0), Wm2_ref[...],
                preferred_element_type=jnp.float32) + bm2_ref[...]
    l3 = (t * INV_SQRT_D).reshape(MAIN_BLK, K, DIM)
    mx = jnp.max(l3, axis=1, keepdims=True)
    e = jnp.exp(l3 - mx)
    s = e / jnp.sum(e, axis=1, keepdims=True)
    z = (Vg + pemix[:, DIM:] + bp2_ref[...]).reshape(MAIN_BLK, K, DIM)
    o = jnp.sum(s * z, axis=1)
    out_ref[...] = (
        jnp.dot(o, Wf_ref[...], preferred_element_type=jnp.float32)
        + bf_ref[...] + x_ref[...])


def _main(A, x, pospad, Gg, Wp1pad, bp1, Wmix, c0, Wm2, bm2, bp2, Wf, bf):
    nb = N // MAIN_BLK
    full = lambda r, c: pl.BlockSpec((r, c), lambda i: (0, 0))
    return pl.pallas_call(
        _main_body,
        grid=(nb,),
        in_specs=[
            pl.BlockSpec((MAIN_BLK, DIM), lambda i: (i, 0)),
            pl.BlockSpec((MAIN_BLK, DIM), lambda i: (i, 0)),
            pl.BlockSpec((MAIN_BLK, PPAD), lambda i: (i, 0)),
            pl.BlockSpec((MAIN_BLK * K, GCOLS), lambda i: (i, 0)),
            full(PPAD, DIM),
            full(1, DIM),
            full(DIM, 2 * DIM),
            full(1, DIM),
            full(DIM, DIM),
            full(1, DIM),
            full(1, DIM),
            full(DIM, DIM),
            full(1, DIM),
        ],
        out_specs=pl.BlockSpec((MAIN_BLK, DIM), lambda i: (i, 0)),
        out_shape=jax.ShapeDtypeStruct((N, DIM), jnp.float32),
    )(A, x, pospad, Gg, Wp1pad, bp1, Wmix, c0, Wm2, bm2, bp2, Wf, bf)


def kernel(x, pos, Wq, Wk, Wv, Wm1, bm1, Wm2, bm2, Wp1, bp1, Wp2, bp2, Wf, bf):
    pospad16 = jnp.pad(pos, ((0, 0), (0, PDP - PD)))
    pospadT = pospad16.T
    pospad = jnp.pad(pos, ((0, 0), (0, PPAD - PD)))
    Wp1pad = jnp.pad(Wp1, ((0, PPAD - PD), (0, 0)))
    r = lambda b: b.reshape(1, DIM)

    A, G, Wmix, c0 = _prep(x, pospad, Wq, Wk, Wv, Wm1, Wp2, r(bm1), r(bp2))
    ind = _knn(pospad16, pospadT)
    Gg = _gather(G, ind.reshape(N * K))
    return _main(A, x, pospad, Gg, Wp1pad, r(bp1), Wmix, c0, Wm2, r(bm2),
                 r(bp2), Wf, r(bf))


# speedup vs baseline: 10.3849x; 1.3774x over previous
import functools

import jax
import jax.numpy as jnp
from jax import lax
from jax.experimental import pallas as pl
from jax.experimental.pallas import tpu as pltpu
from jax.experimental.pallas import tpu_sc as plsc

N, DIM, PD, K = 4096, 256, 3, 16
PDP = 16
PPAD = 128
GCOLS = DIM + PPAD
KNN_BLK = 256
MAIN_BLK = 128
NW = 32
ROWS_PER_W = (N * K) // NW
GCHUNK = 128
INV_SQRT_D = 1.0 / 16.0
H = DIM // 2


def _pack(a):
    bits = lax.bitcast_convert_type(a, jnp.int32)
    hi = jnp.bitwise_and(bits[:, :H], jnp.int32(-65536))
    lo = lax.shift_right_logical(bits[:, H:], 16)
    return lax.bitcast_convert_type(jnp.bitwise_or(hi, lo), jnp.float32)


def _unpack(p):
    bits = lax.bitcast_convert_type(p, jnp.int32)
    hi = lax.bitcast_convert_type(
        jnp.bitwise_and(bits, jnp.int32(-65536)), jnp.float32)
    lo = lax.bitcast_convert_type(
        lax.shift_left(bits, 16), jnp.float32)
    return jnp.concatenate([hi, lo], axis=1)


def _prep_body(x_ref, pospad_ref, Wq_ref, Wk_ref, Wv_ref, Wm1_ref, Wp2_ref,
               bm1_ref, bp2_ref, A_ref, G_ref, Wmix_ref, c0_ref):
    x = x_ref[...]
    Wm1 = Wm1_ref[...]
    A_ref[...] = x @ (Wq_ref[...] @ Wm1)
    kx = (x @ (Wk_ref[...] @ Wm1)).astype(jnp.bfloat16).astype(jnp.float32)
    v = (x @ Wv_ref[...]).astype(jnp.bfloat16).astype(jnp.float32)
    G_ref[...] = jnp.concatenate(
        [_pack(kx), _pack(v), pospad_ref[...]], axis=1)
    Wp2 = Wp2_ref[...]
    Wmix_ref[...] = jnp.concatenate(
        [Wp2 @ Wm1, Wp2], axis=1).astype(jnp.bfloat16)
    c0_ref[...] = bp2_ref[...] @ Wm1 + bm1_ref[...]


def _prep(x, pospad, Wq, Wk, Wv, Wm1, Wp2, bm1, bp2):
    return pl.pallas_call(
        _prep_body,
        out_shape=(
            jax.ShapeDtypeStruct((N, DIM), jnp.float32),
            jax.ShapeDtypeStruct((N, GCOLS), jnp.float32),
            jax.ShapeDtypeStruct((DIM, 2 * DIM), jnp.bfloat16),
            jax.ShapeDtypeStruct((1, DIM), jnp.float32),
        ),
    )(x, pospad, Wq, Wk, Wv, Wm1, Wp2, bm1, bp2)


def _knn_body(posb_ref, pospadT_ref, ind_ref):
    posb = posb_ref[...]
    pT = pospadT_ref[...]
    sqb = jnp.sum(posb * posb, axis=1, keepdims=True)
    sqf = jnp.sum(pT * pT, axis=0, keepdims=True)
    d2 = sqb + sqf - 2.0 * jnp.dot(posb, pT, preferred_element_type=jnp.float32)
    bits = lax.bitcast_convert_type(jnp.maximum(d2, 0.0), jnp.int32)
    iota = lax.broadcasted_iota(jnp.int32, (KNN_BLK, N), 1)
    keys = jnp.bitwise_or(jnp.bitwise_and(bits, jnp.int32(-4096)), iota)
    imax = jnp.int32(2147483647)
    cols = []
    for _ in range(K):
        mk = jnp.min(keys, axis=1, keepdims=True)
        cols.append(jnp.bitwise_and(mk, jnp.int32(4095)))
        keys = jnp.where(keys == mk, imax, keys)
    ind_ref[...] = jnp.concatenate(cols, axis=1)


def _knn(pospad, pospadT):
    return pl.pallas_call(
        _knn_body,
        grid=(N // KNN_BLK,),
        in_specs=[
            pl.BlockSpec((KNN_BLK, PDP), lambda i: (i, 0)),
            pl.BlockSpec((PDP, N), lambda i: (0, 0)),
        ],
        out_specs=pl.BlockSpec((KNN_BLK, K), lambda i: (i, 0)),
        out_shape=jax.ShapeDtypeStruct((N, K), jnp.int32),
    )(pospad, pospadT)


def _gather_body(G_hbm, ind_hbm, Gg_hbm, idx_v, rows_v, sem):
    wid = lax.axis_index("s") * 2 + lax.axis_index("c")
    base = wid * ROWS_PER_W
    pltpu.sync_copy(ind_hbm.at[pl.ds(base, ROWS_PER_W)], idx_v)
    for c in range(ROWS_PER_W // GCHUNK):
        pltpu.async_copy(
            G_hbm.at[idx_v.at[pl.ds(c * GCHUNK, GCHUNK)]], rows_v, sem
        ).wait()
        pltpu.sync_copy(rows_v, Gg_hbm.at[pl.ds(base + c * GCHUNK, GCHUNK)])


def _gather(G, ind_flat):
    mesh = plsc.VectorSubcoreMesh(core_axis_name="c", subcore_axis_name="s")
    run = pl.kernel(
        _gather_body,
        mesh=mesh,
        out_type=jax.ShapeDtypeStruct((N * K, GCOLS), jnp.float32),
        scratch_types=[
            pltpu.VMEM((ROWS_PER_W,), jnp.int32),
            pltpu.VMEM((GCHUNK, GCOLS), jnp.float32),
            pltpu.SemaphoreType.DMA,
        ],
    )
    return run(G, ind_flat)


def _main_body(A_ref, x_ref, posb_ref, Gg_ref, Wp1_ref, bp1_ref, Wmix_ref,
               c0_ref, Wm2_ref, bm2_ref, bp2_ref, Wf_ref, bf_ref, out_ref):
    Gg = Gg_ref[...]
    Bg = _unpack(Gg[:, :H])
    Vg = _unpack(Gg[:, H:DIM])
    posg = Gg[:, DIM:]
    posb = posb_ref[...]
    relb = jnp.broadcast_to(
        posb.reshape(MAIN_BLK, 1, PPAD), (MAIN_BLK, K, PPAD)
    ).reshape(MAIN_BLK * K, PPAD)
    rel = (relb - posg).astype(jnp.bfloat16)
    h = jnp.maximum(
        jnp.dot(rel, Wp1_ref[...], preferred_element_type=jnp.float32)
        + bp1_ref[...], 0.0).astype(jnp.bfloat16)
    pemix = jnp.dot(h, Wmix_ref[...], preferred_element_type=jnp.float32)
    a = jnp.broadcast_to(
        A_ref[...].reshape(MAIN_BLK, 1, DIM), (MAIN_BLK, K, DIM)
    ).reshape(MAIN_BLK * K, DIM)
    u = a - Bg + pemix[:, :DIM] + c0_ref[...]
    t = jnp.dot(jnp.maximum(u, 0.0).astype(jnp.bfloat16),
                Wm2_ref[...].astype(jnp.bfloat16),
                preferred_element_type=jnp.float32) + bm2_ref[...]
    l3 = (t * INV_SQRT_D).reshape(MAIN_BLK, K, DIM)
    mx = jnp.max(l3, axis=1, keepdims=True)
    e = jnp.exp(l3 - mx)
    s = e / jnp.sum(e, axis=1, keepdims=True)
    z = (Vg + pemix[:, DIM:] + bp2_ref[...]).reshape(MAIN_BLK, K, DIM)
    o = jnp.sum(s * z, axis=1)
    out_ref[...] = (
        jnp.dot(o.astype(jnp.bfloat16), Wf_ref[...].astype(jnp.bfloat16),
                preferred_element_type=jnp.float32)
        + bf_ref[...] + x_ref[...])


def _main(A, x, pospad, Gg, Wp1pad, bp1, Wmix, c0, Wm2, bm2, bp2, Wf, bf):
    nb = N // MAIN_BLK
    full = lambda r, c: pl.BlockSpec((r, c), lambda i: (0, 0))
    return pl.pallas_call(
        _main_body,
        grid=(nb,),
        in_specs=[
            pl.BlockSpec((MAIN_BLK, DIM), lambda i: (i, 0)),
            pl.BlockSpec((MAIN_BLK, DIM), lambda i: (i, 0)),
            pl.BlockSpec((MAIN_BLK, PPAD), lambda i: (i, 0)),
            pl.BlockSpec((MAIN_BLK * K, GCOLS), lambda i: (i, 0)),
            full(PPAD, DIM),
            full(1, DIM),
            full(DIM, 2 * DIM),
            full(1, DIM),
            full(DIM, DIM),
            full(1, DIM),
            full(1, DIM),
            full(DIM, DIM),
            full(1, DIM),
        ],
        out_specs=pl.BlockSpec((MAIN_BLK, DIM), lambda i: (i, 0)),
        out_shape=jax.ShapeDtypeStruct((N, DIM), jnp.float32),
    )(A, x, pospad, Gg, Wp1pad, bp1, Wmix, c0, Wm2, bm2, bp2, Wf, bf)


def kernel(x, pos, Wq, Wk, Wv, Wm1, bm1, Wm2, bm2, Wp1, bp1, Wp2, bp2, Wf, bf):
    pospad16 = jnp.pad(pos, ((0, 0), (0, PDP - PD)))
    pospadT = pospad16.T
    pospad = jnp.pad(pos, ((0, 0), (0, PPAD - PD)))
    Wp1pad = jnp.pad(Wp1, ((0, PPAD - PD), (0, 0))).astype(jnp.bfloat16)
    r = lambda b: b.reshape(1, DIM)

    A, G, Wmix, c0 = _prep(x, pospad, Wq, Wk, Wv, Wm1, Wp2, r(bm1), r(bp2))
    ind = _knn(pospad16, pospadT)
    Gg = _gather(G, ind.reshape(N * K))
    return _main(A, x, pospad, Gg, Wp1pad, r(bp1), Wmix, c0, Wm2, r(bm2),
                 r(bp2), Wf, r(bf))


# speedup vs baseline: 12.8631x; 1.2386x over previous
import functools

import jax
import jax.numpy as jnp
from jax import lax
from jax.experimental import pallas as pl
from jax.experimental.pallas import tpu as pltpu
from jax.experimental.pallas import tpu_sc as plsc

N, DIM, PD, K = 4096, 256, 3, 16
PDP = 16
PPAD = 128
GCOLS = DIM + PPAD
KNN_BLK = 256
MAIN_BLK = 128
NW = 32
ROWS_PER_W = (N * K) // NW
GCHUNK = 128
INV_SQRT_D = 1.0 / 16.0
H = DIM // 2


def _pack(a):
    bits = lax.bitcast_convert_type(a, jnp.int32)
    hi = jnp.bitwise_and(bits[:, :H], jnp.int32(-65536))
    lo = lax.shift_right_logical(bits[:, H:], 16)
    return lax.bitcast_convert_type(jnp.bitwise_or(hi, lo), jnp.float32)


def _unpack(p):
    bits = lax.bitcast_convert_type(p, jnp.int32)
    hi = lax.bitcast_convert_type(
        jnp.bitwise_and(bits, jnp.int32(-65536)), jnp.float32)
    lo = lax.bitcast_convert_type(
        lax.shift_left(bits, 16), jnp.float32)
    return jnp.concatenate([hi, lo], axis=1)


def _prep_body(x_ref, pospad_ref, Wq_ref, Wk_ref, Wv_ref, Wm1_ref, Wp2_ref,
               bm1_ref, bp2_ref, A_ref, G_ref, Wmix_ref, c0_ref):
    x = x_ref[...]
    Wm1 = Wm1_ref[...]
    A_ref[...] = x @ (Wq_ref[...] @ Wm1)
    kx = (x @ (Wk_ref[...] @ Wm1)).astype(jnp.bfloat16).astype(jnp.float32)
    v = (x @ Wv_ref[...]).astype(jnp.bfloat16).astype(jnp.float32)
    G_ref[...] = jnp.concatenate(
        [_pack(kx), _pack(v), pospad_ref[...]], axis=1)
    Wp2 = Wp2_ref[...]
    Wmix_ref[...] = jnp.concatenate(
        [Wp2 @ Wm1, Wp2], axis=1).astype(jnp.bfloat16)
    c0_ref[...] = bp2_ref[...] @ Wm1 + bm1_ref[...]


def _prep(x, pospad, Wq, Wk, Wv, Wm1, Wp2, bm1, bp2):
    return pl.pallas_call(
        _prep_body,
        out_shape=(
            jax.ShapeDtypeStruct((N, DIM), jnp.float32),
            jax.ShapeDtypeStruct((N, GCOLS), jnp.float32),
            jax.ShapeDtypeStruct((DIM, 2 * DIM), jnp.bfloat16),
            jax.ShapeDtypeStruct((1, DIM), jnp.float32),
        ),
    )(x, pospad, Wq, Wk, Wv, Wm1, Wp2, bm1, bp2)


def _batcher_pairs(n):
    pairs = []
    p = 1
    while p < n:
        k = p
        while k >= 1:
            for j in range(k % p, n - k, 2 * k):
                for i in range(0, min(k, n - j - k)):
                    if (i + j) // (2 * p) == (i + j + k) // (2 * p):
                        pairs.append((i + j, i + j + k))
            k //= 2
        p *= 2
    return pairs


_SORT16 = _batcher_pairs(K)


def _knn_body(posb_ref, pospadT_ref, ind_ref):
    posb = posb_ref[...]
    pT = pospadT_ref[...]
    sqb = jnp.sum(posb * posb, axis=1, keepdims=True)
    sqf = jnp.sum(pT * pT, axis=0, keepdims=True)
    d2 = sqb + sqf - 2.0 * jnp.dot(posb, pT, preferred_element_type=jnp.float32)
    bits = lax.bitcast_convert_type(jnp.maximum(d2, 0.0), jnp.int32)
    iota = lax.broadcasted_iota(jnp.int32, (KNN_BLK, N), 1)
    keys = jnp.bitwise_or(jnp.bitwise_and(bits, jnp.int32(-4096)), iota)
    imax = jnp.int32(2147483647)
    ng = N // K
    S = [keys[:, i * ng:(i + 1) * ng] for i in range(K)]
    for a, b in _SORT16:
        lo = jnp.minimum(S[a], S[b])
        hi = jnp.maximum(S[a], S[b])
        S[a], S[b] = lo, hi
    cols = []
    for _ in range(K):
        mk = jnp.min(S[0], axis=1, keepdims=True)
        cols.append(jnp.bitwise_and(mk, jnp.int32(4095)))
        m = S[0] == mk
        for i in range(K - 1):
            S[i] = jnp.where(m, S[i + 1], S[i])
        S[K - 1] = jnp.where(m, imax, S[K - 1])
    ind_ref[...] = jnp.concatenate(cols, axis=1)


def _knn(pospad, pospadT):
    return pl.pallas_call(
        _knn_body,
        grid=(N // KNN_BLK,),
        in_specs=[
            pl.BlockSpec((KNN_BLK, PDP), lambda i: (i, 0)),
            pl.BlockSpec((PDP, N), lambda i: (0, 0)),
        ],
        out_specs=pl.BlockSpec((KNN_BLK, K), lambda i: (i, 0)),
        out_shape=jax.ShapeDtypeStruct((N, K), jnp.int32),
    )(pospad, pospadT)


def _gather_body(G_hbm, ind_hbm, Gg_hbm, idx_v, rows0, rows1, sem0, sem1):
    wid = lax.axis_index("s") * 2 + lax.axis_index("c")
    base = wid * ROWS_PER_W
    pltpu.sync_copy(ind_hbm.at[pl.ds(base, ROWS_PER_W)], idx_v)
    bufs = (rows0, rows1)
    sems = (sem0, sem1)
    nch = ROWS_PER_W // GCHUNK
    cps = [None, None]
    cps[0] = pltpu.async_copy(
        G_hbm.at[idx_v.at[pl.ds(0, GCHUNK)]], bufs[0], sems[0])
    for c in range(nch):
        if c + 1 < nch:
            p = (c + 1) % 2
            cps[p] = pltpu.async_copy(
                G_hbm.at[idx_v.at[pl.ds((c + 1) * GCHUNK, GCHUNK)]],
                bufs[p], sems[p])
        cps[c % 2].wait()
        pltpu.sync_copy(bufs[c % 2],
                        Gg_hbm.at[pl.ds(base + c * GCHUNK, GCHUNK)])


def _gather(G, ind_flat):
    mesh = plsc.VectorSubcoreMesh(core_axis_name="c", subcore_axis_name="s")
    run = pl.kernel(
        _gather_body,
        mesh=mesh,
        out_type=jax.ShapeDtypeStruct((N * K, GCOLS), jnp.float32),
        scratch_types=[
            pltpu.VMEM((ROWS_PER_W,), jnp.int32),
            pltpu.VMEM((GCHUNK, GCOLS), jnp.float32),
            pltpu.VMEM((GCHUNK, GCOLS), jnp.float32),
            pltpu.SemaphoreType.DMA,
            pltpu.SemaphoreType.DMA,
        ],
    )
    return run(G, ind_flat)


def _main_body(A_ref, x_ref, posb_ref, Gg_ref, Wp1_ref, bp1_ref, Wmix_ref,
               c0_ref, Wm2_ref, bm2_ref, bp2_ref, Wf_ref, bf_ref, out_ref):
    Gg = Gg_ref[...]
    Bg = _unpack(Gg[:, :H])
    Vg = _unpack(Gg[:, H:DIM])
    posg = Gg[:, DIM:]
    posb = posb_ref[...]
    relb = jnp.broadcast_to(
        posb.reshape(MAIN_BLK, 1, PPAD), (MAIN_BLK, K, PPAD)
    ).reshape(MAIN_BLK * K, PPAD)
    rel = (relb - posg).astype(jnp.bfloat16)
    h = jnp.maximum(
        jnp.dot(rel, Wp1_ref[...], preferred_element_type=jnp.float32)
        + bp1_ref[...], 0.0).astype(jnp.bfloat16)
    pemix = jnp.dot(h, Wmix_ref[...], preferred_element_type=jnp.float32)
    a = jnp.broadcast_to(
        A_ref[...].reshape(MAIN_BLK, 1, DIM), (MAIN_BLK, K, DIM)
    ).reshape(MAIN_BLK * K, DIM)
    u = a - Bg + pemix[:, :DIM] + c0_ref[...]
    t = jnp.dot(jnp.maximum(u, 0.0).astype(jnp.bfloat16),
                Wm2_ref[...].astype(jnp.bfloat16),
                preferred_element_type=jnp.float32) + bm2_ref[...]
    l3 = (t * INV_SQRT_D).reshape(MAIN_BLK, K, DIM)
    mx = jnp.max(l3, axis=1, keepdims=True)
    e = jnp.exp(l3 - mx)
    s = e / jnp.sum(e, axis=1, keepdims=True)
    z = (Vg + pemix[:, DIM:] + bp2_ref[...]).reshape(MAIN_BLK, K, DIM)
    o = jnp.sum(s * z, axis=1)
    out_ref[...] = (
        jnp.dot(o.astype(jnp.bfloat16), Wf_ref[...].astype(jnp.bfloat16),
                preferred_element_type=jnp.float32)
        + bf_ref[...] + x_ref[...])


def _main(A, x, pospad, Gg, Wp1pad, bp1, Wmix, c0, Wm2, bm2, bp2, Wf, bf):
    nb = N // MAIN_BLK
    full = lambda r, c: pl.BlockSpec((r, c), lambda i: (0, 0))
    return pl.pallas_call(
        _main_body,
        grid=(nb,),
        in_specs=[
            pl.BlockSpec((MAIN_BLK, DIM), lambda i: (i, 0)),
            pl.BlockSpec((MAIN_BLK, DIM), lambda i: (i, 0)),
            pl.BlockSpec((MAIN_BLK, PPAD), lambda i: (i, 0)),
            pl.BlockSpec((MAIN_BLK * K, GCOLS), lambda i: (i, 0)),
            full(PPAD, DIM),
            full(1, DIM),
            full(DIM, 2 * DIM),
            full(1, DIM),
            full(DIM, DIM),
            full(1, DIM),
            full(1, DIM),
            full(DIM, DIM),
            full(1, DIM),
        ],
        out_specs=pl.BlockSpec((MAIN_BLK, DIM), lambda i: (i, 0)),
        out_shape=jax.ShapeDtypeStruct((N, DIM), jnp.float32),
    )(A, x, pospad, Gg, Wp1pad, bp1, Wmix, c0, Wm2, bm2, bp2, Wf, bf)


def kernel(x, pos, Wq, Wk, Wv, Wm1, bm1, Wm2, bm2, Wp1, bp1, Wp2, bp2, Wf, bf):
    pospad16 = jnp.pad(pos, ((0, 0), (0, PDP - PD)))
    pospadT = pospad16.T
    pospad = jnp.pad(pos, ((0, 0), (0, PPAD - PD)))
    Wp1pad = jnp.pad(Wp1, ((0, PPAD - PD), (0, 0))).astype(jnp.bfloat16)
    r = lambda b: b.reshape(1, DIM)

    A, G, Wmix, c0 = _prep(x, pospad, Wq, Wk, Wv, Wm1, Wp2, r(bm1), r(bp2))
    ind = _knn(pospad16, pospadT)
    Gg = _gather(G, ind.reshape(N * K))
    return _main(A, x, pospad, Gg, Wp1pad, r(bp1), Wmix, c0, Wm2, r(bm2),
                 r(bp2), Wf, r(bf))
